# baseline (speedup 1.0000x reference)
import functools

import jax
import jax.numpy as jnp
from jax.experimental import pallas as pl
from jax.experimental.pallas import tpu as pltpu

_N = 4096
_H = 128
_Hh = _H // 2
_R = 2

_BM = 256
_BH = _BM // 2
_GM = _N // _BM


def _bigcn_kernel(inps_ref, fw00, fw01, fw10, fw11, bw00, bw01, bw10, bw11,
                  Wfw_ref, bfw_ref, Wbw_ref, bbw_ref, W1_ref, b1_ref,
                  out_ref, h_ref):
    m = pl.program_id(0)

    @pl.when(m == 0)
    def _project():
        x = inps_ref[...]
        for i in range(_R):
            h_ref[:, i * _H:i * _H + _Hh] = jnp.dot(
                x, Wbw_ref[i], preferred_element_type=jnp.float32)
            h_ref[:, i * _H + _Hh:(i + 1) * _H] = jnp.dot(
                x, Wfw_ref[i], preferred_element_type=jnp.float32)

    bias = jnp.concatenate(
        [jnp.sum(bbw_ref[...], axis=0), jnp.sum(bfw_ref[...], axis=0)])

    halves = (((bw00, bw10), (fw00, fw10)), ((bw01, bw11), (fw01, fw11)))
    for hrow, (bws, fws) in enumerate(halves):
        left = jnp.dot(bws[0][0], h_ref[:, :_Hh],
                       preferred_element_type=jnp.float32)
        right = jnp.dot(fws[0][0], h_ref[:, _Hh:_H],
                        preferred_element_type=jnp.float32)
        for i in range(1, _R):
            left = left + jnp.dot(bws[i][0], h_ref[:, i * _H:i * _H + _Hh],
                                  preferred_element_type=jnp.float32)
            right = right + jnp.dot(fws[i][0],
                                    h_ref[:, i * _H + _Hh:(i + 1) * _H],
                                    preferred_element_type=jnp.float32)

        s = jnp.maximum(
            jnp.concatenate([left, right], axis=1) + bias[None, :], 0.0)
        feats = jax.lax.dot_general(
            s, W1_ref[...], (((1,), (1,)), ((), ())),
            preferred_element_type=jnp.float32)
        rows = pl.ds(hrow * _BH, _BH)
        out_ref[rows, :] = feats + b1_ref[...][None, :] + \
            inps_ref[pl.ds(m * _BM + hrow * _BH, _BH), :]


def _adj_spec(r, h):
    return pl.BlockSpec((1, _BH, _N), lambda m, r=r, h=h: (r, 2 * m + h, 0))


@functools.partial(jax.jit, static_argnames=())
def kernel(inps, fw_adjs, bw_adjs, W_fw, b_fw, W_bw, b_bw, W1, b1):
    adj_specs = [_adj_spec(r, h) for r in (0, 1) for h in (0, 1)]
    return pl.pallas_call(
        _bigcn_kernel,
        grid=(_GM,),
        in_specs=(
            [pl.BlockSpec((_N, _H), lambda m: (0, 0))]
            + adj_specs
            + adj_specs
            + [
                pl.BlockSpec((_R, _H, _Hh), lambda m: (0, 0, 0)),
                pl.BlockSpec((_R, _Hh), lambda m: (0, 0)),
                pl.BlockSpec((_R, _H, _Hh), lambda m: (0, 0, 0)),
                pl.BlockSpec((_R, _Hh), lambda m: (0, 0)),
                pl.BlockSpec((_H, _H), lambda m: (0, 0)),
                pl.BlockSpec((_H,), lambda m: (0,)),
            ]
        ),
        out_specs=pl.BlockSpec((_BM, _H), lambda m: (m, 0)),
        out_shape=jax.ShapeDtypeStruct((_N, _H), jnp.float32),
        scratch_shapes=[pltpu.VMEM((_N, _R * _H), jnp.float32)],
    )(inps,
      fw_adjs, fw_adjs, fw_adjs, fw_adjs,
      bw_adjs, bw_adjs, bw_adjs, bw_adjs,
      W_fw, b_fw, W_bw, b_bw, W1, b1)


# speedup vs baseline: 1.0263x; 1.0263x over previous
import functools

import jax
import jax.numpy as jnp
from jax.experimental import pallas as pl
from jax.experimental.pallas import tpu as pltpu

_N = 4096
_H = 128
_Hh = _H // 2
_R = 2

_BM = 256
_GM = _N // _BM


def _bigcn_kernel(inps_ref, fw_ref, bw_ref, Wfw_ref, bfw_ref, Wbw_ref,
                  bbw_ref, W1_ref, b1_ref, out_ref, h_ref):
    m = pl.program_id(0)

    @pl.when(m == 0)
    def _project():
        x = inps_ref[...]
        for i in range(_R):
            h_ref[:, i * _H:i * _H + _Hh] = jnp.dot(
                x, Wbw_ref[i], preferred_element_type=jnp.float32)
            h_ref[:, i * _H + _Hh:(i + 1) * _H] = jnp.dot(
                x, Wfw_ref[i], preferred_element_type=jnp.float32)

    bias = jnp.concatenate(
        [jnp.sum(bbw_ref[...], axis=0), jnp.sum(bfw_ref[...], axis=0)])

    left = jnp.dot(bw_ref[0], h_ref[:, :_Hh],
                   preferred_element_type=jnp.float32)
    right = jnp.dot(fw_ref[0], h_ref[:, _Hh:_H],
                    preferred_element_type=jnp.float32)
    for i in range(1, _R):
        left = left + jnp.dot(bw_ref[i], h_ref[:, i * _H:i * _H + _Hh],
                              preferred_element_type=jnp.float32)
        right = right + jnp.dot(fw_ref[i],
                                h_ref[:, i * _H + _Hh:(i + 1) * _H],
                                preferred_element_type=jnp.float32)

    s = jnp.maximum(
        jnp.concatenate([left, right], axis=1) + bias[None, :], 0.0)
    feats = jax.lax.dot_general(
        s, W1_ref[...], (((1,), (1,)), ((), ())),
        preferred_element_type=jnp.float32)
    out_ref[...] = feats + b1_ref[...][None, :] + \
        inps_ref[pl.ds(m * _BM, _BM), :]


@functools.partial(jax.jit, static_argnames=())
def kernel(inps, fw_adjs, bw_adjs, W_fw, b_fw, W_bw, b_bw, W1, b1):
    return pl.pallas_call(
        _bigcn_kernel,
        grid=(_GM,),
        in_specs=[
            pl.BlockSpec((_N, _H), lambda m: (0, 0)),
            pl.BlockSpec((_R, _BM, _N), lambda m: (0, m, 0)),
            pl.BlockSpec((_R, _BM, _N), lambda m: (0, m, 0)),
            pl.BlockSpec((_R, _H, _Hh), lambda m: (0, 0, 0)),
            pl.BlockSpec((_R, _Hh), lambda m: (0, 0)),
            pl.BlockSpec((_R, _H, _Hh), lambda m: (0, 0, 0)),
            pl.BlockSpec((_R, _Hh), lambda m: (0, 0)),
            pl.BlockSpec((_H, _H), lambda m: (0, 0)),
            pl.BlockSpec((_H,), lambda m: (0,)),
        ],
        out_specs=pl.BlockSpec((_BM, _H), lambda m: (m, 0)),
        out_shape=jax.ShapeDtypeStruct((_N, _H), jnp.float32),
        scratch_shapes=[pltpu.VMEM((_N, _R * _H), jnp.float32)],
    )(inps, fw_adjs, bw_adjs, W_fw, b_fw, W_bw, b_bw, W1, b1)
